# baseline (speedup 1.0000x reference)
import jax
import jax.numpy as jnp
from jax import lax
from jax.experimental import pallas as pl
from jax.experimental.pallas import tpu as pltpu


def _ceil_to(n, m):
    return ((n + m - 1) // m) * m


_TILE = 512


def _fwd_body(x_ref, t1_ref, b1_ref, t2_ref, b2_ref,
              w1_ref, fb1_ref, w2_ref, fb2_ref, w3_ref, fb3_ref,
              o_ref, xf_ref):
    q1 = t1_ref.shape[1] // 4
    q2 = t2_ref.shape[1] // 4

    i = pl.program_id(0)
    cur = lax.rem(i, 2)
    prev = 1 - cur

    xc = xf_ref[prev]

    d = jnp.dot(xc, t1_ref[...], preferred_element_type=jnp.float32)
    m = jnp.maximum(jnp.maximum(d[:, :q1], d[:, q1:2 * q1]),
                    jnp.maximum(d[:, 2 * q1:3 * q1], d[:, 3 * q1:]))
    h = jnp.maximum(m + b1_ref[...], 0.0).astype(jnp.bfloat16)

    e = jnp.dot(h, t2_ref[...], preferred_element_type=jnp.float32)
    m2 = jnp.maximum(jnp.maximum(e[:, :q2], e[:, q2:2 * q2]),
                     jnp.maximum(e[:, 2 * q2:3 * q2], e[:, 3 * q2:]))
    g = jnp.maximum(m2 + b2_ref[...], 0.0).astype(jnp.bfloat16)

    z = jnp.dot(g, w1_ref[...], preferred_element_type=jnp.float32)
    z = jnp.maximum(z + fb1_ref[...], 0.0).astype(jnp.bfloat16)
    z = jnp.dot(z, w2_ref[...], preferred_element_type=jnp.float32)
    z = jnp.maximum(z + fb2_ref[...], 0.0).astype(jnp.bfloat16)
    o = jnp.dot(z, w3_ref[...], preferred_element_type=jnp.float32)
    o_ref[...] = o + fb3_ref[...]

    xf_ref[cur] = x_ref[...].astype(jnp.bfloat16).reshape(_TILE, 784)


def kernel(x, t1, b1, t2, b2, w1, fb1, w2, fb2, w3, fb3):
    N = x.shape[0]
    x3 = x.reshape(N, 28, 28)

    padded = _ceil_to(N, _TILE)
    if padded != N:
        x3 = jnp.pad(x3, ((0, padded - N), (0, 0), (0, 0)))
    nt = padded // _TILE

    ncp = fb3.shape[-1]
    const = lambda a: pl.BlockSpec(a.shape, (lambda i: (0,) * a.ndim),
                                   pipeline_mode=pl.Buffered(1))

    out = pl.pallas_call(
        _fwd_body,
        out_shape=jax.ShapeDtypeStruct((padded, ncp), jnp.float32),
        grid=(nt + 1,),
        in_specs=[
            pl.BlockSpec((_TILE, 28, 28),
                         lambda i: (jnp.minimum(i, nt - 1), 0, 0)),
            const(t1), const(b1), const(t2), const(b2),
            const(w1), const(fb1), const(w2), const(fb2),
            const(w3), const(fb3),
        ],
        out_specs=pl.BlockSpec((_TILE, ncp),
                               lambda i: (jnp.maximum(i, 1) - 1, 0)),
        scratch_shapes=[pltpu.VMEM((2, _TILE, 784), jnp.bfloat16)],
        compiler_params=pltpu.CompilerParams(
            dimension_semantics=("arbitrary",),
            vmem_limit_bytes=56 * 1024 * 1024,
        ),
    )(x3, t1, b1, t2, b2, w1, fb1, w2, fb2, w3, fb3)
    return out[:N, :10]
